# Initial kernel scaffold; baseline (speedup 1.0000x reference)
import jax
import jax.numpy as jnp
from jax.experimental import pallas as pl


def kernel(pred, gt):
    raise NotImplementedError("write your pallas kernel here")


# speedup vs baseline: 1.2705x; 1.2705x over previous
import jax
import jax.numpy as jnp
from jax.experimental import pallas as pl
from jax.experimental.pallas import tpu as pltpu

_BLOCK_ROWS = 1024


def _loss_kernel(pred_ref, gt_ref, out_ref, acc_ref):
    i = pl.program_id(0)

    @pl.when(i == 0)
    def _():
        acc_ref[0] = 0.0

    g = gt_ref[...]
    d = pred_ref[...] - g
    mask = (g[:, 0:1] != -1.0).astype(jnp.float32)
    acc_ref[0] += jnp.sum(d * d * mask)

    @pl.when(i == pl.num_programs(0) - 1)
    def _():
        out_ref[0, 0] = acc_ref[0]


def kernel(pred, gt):
    B, N, C = pred.shape
    rows = B * N
    pred2 = pred.reshape(rows, C)
    gt2 = gt.reshape(rows, C)
    grid = rows // _BLOCK_ROWS

    out = pl.pallas_call(
        _loss_kernel,
        grid=(grid,),
        in_specs=[
            pl.BlockSpec((_BLOCK_ROWS, C), lambda i: (i, 0)),
            pl.BlockSpec((_BLOCK_ROWS, C), lambda i: (i, 0)),
        ],
        out_specs=pl.BlockSpec((1, 1), lambda i: (0, 0), memory_space=pltpu.SMEM),
        out_shape=jax.ShapeDtypeStruct((1, 1), jnp.float32),
        scratch_shapes=[pltpu.SMEM((1,), jnp.float32)],
    )(pred2, gt2)
    return out[0, 0]


# speedup vs baseline: 1.5263x; 1.2014x over previous
import jax
import jax.numpy as jnp
from jax.experimental import pallas as pl
from jax.experimental.pallas import tpu as pltpu

_BLOCK_ROWS = 2048


def _loss_kernel(pred_ref, gt_ref, out_ref, acc_ref):
    i = pl.program_id(0)

    @pl.when(i == 0)
    def _():
        acc_ref[0] = 0.0

    g = gt_ref[...]
    d = pred_ref[...] - g
    mask = (g[:, 0:1] != -1.0).astype(jnp.float32)
    acc_ref[0] += jnp.sum(d * d * mask)

    @pl.when(i == pl.num_programs(0) - 1)
    def _():
        out_ref[0, 0] = acc_ref[0]


def kernel(pred, gt):
    B, N, C = pred.shape
    rows = B * N
    pred2 = pred.reshape(rows, C)
    gt2 = gt.reshape(rows, C)
    grid = rows // _BLOCK_ROWS

    out = pl.pallas_call(
        _loss_kernel,
        grid=(grid,),
        in_specs=[
            pl.BlockSpec((_BLOCK_ROWS, C), lambda i: (i, 0)),
            pl.BlockSpec((_BLOCK_ROWS, C), lambda i: (i, 0)),
        ],
        out_specs=pl.BlockSpec((1, 1), lambda i: (0, 0), memory_space=pltpu.SMEM),
        out_shape=jax.ShapeDtypeStruct((1, 1), jnp.float32),
        scratch_shapes=[pltpu.SMEM((1,), jnp.float32)],
    )(pred2, gt2)
    return out[0, 0]
